# baseline (speedup 1.0000x reference)
import functools

import jax
import jax.numpy as jnp
from jax import lax
from jax.experimental import pallas as pl
from jax.experimental.pallas import tpu as pltpu
from jax.experimental.pallas import tpu_sc as plsc

VOCAB, D, H, C = 1000000, 32, 32, 10
B, L = 4096, 50

NUM_CORES = 2
NUM_SUBCORES = 16
NW = NUM_CORES * NUM_SUBCORES
SPW = B // NW
SR = 128
VPAD = 1000064
NC4 = 4
SW = NC4 * D * 16
K = 2
NR = SPW // K

_mesh = plsc.VectorSubcoreMesh(core_axis_name="c", subcore_axis_name="s")


def _tree_sum(vals):
    vals = list(vals)
    while len(vals) > 1:
        nxt = [vals[i] + vals[i + 1] for i in range(0, len(vals) - 1, 2)]
        if len(vals) % 2:
            nxt.append(vals[-1])
        vals = nxt
    return vals[0]


@functools.partial(
    pl.kernel,
    mesh=_mesh,
    compiler_params=pltpu.CompilerParams(use_tc_tiling_on_sc=False, needs_layout_passes=False),
    out_type=jax.ShapeDtypeStruct((B, D), jnp.float32),
    scratch_types=[
        pltpu.VMEM((SPW, SR), jnp.int32),
        pltpu.VMEM((K, SW), jnp.float32),
        pltpu.VMEM((K, SW), jnp.float32),
        pltpu.VMEM((SW,), jnp.float32),
        pltpu.VMEM((K, D), jnp.float32),
        pltpu.VMEM((K, D), jnp.float32),
        pltpu.SemaphoreType.DMA,
        pltpu.SemaphoreType.DMA,
        pltpu.SemaphoreType.DMA,
        pltpu.SemaphoreType.DMA,
    ],
)
def _sc_gather_sum(xp_hbm, table_hbm, out_hbm,
                   idx_v, stage_a, stage_b, rowbuf, out_a, out_b,
                   sem_a, sem_b, sem_oa, sem_ob):
    wid = lax.axis_index("s") * NUM_CORES + lax.axis_index("c")
    sbase = wid * SPW
    pltpu.sync_copy(xp_hbm.at[pl.ds(sbase, SPW)], idx_v)

    lane = lax.iota(jnp.int32, 16)
    pos_base = lane * D

    def issue(stage, sem, r):
        @pl.when(r < NR)
        def _():
            for j in range(K):
                s = r * K + j
                for c in range(NC4):
                    vc = idx_v[s, pl.ds(16 * c, 16)]
                    for d in range(D):
                        iv = vc + d * VPAD
                        pltpu.async_copy(
                            table_hbm.at[iv],
                            stage.at[j, pl.ds((c * D + d) * 16, 16)],
                            sem)

    def drain(stage, sem):
        for j in range(K):
            pltpu.make_async_copy(
                table_hbm.at[pl.ds(0, SW)], stage.at[j], sem).wait()

    def consume(stage, out_buf):
        for j in range(K):
            for c in range(NC4):
                for d in range(D):
                    v16 = stage[j, pl.ds((c * D + d) * 16, 16)]
                    pos = pos_base + (c * 16 * D + d)
                    plsc.store_scatter(rowbuf, [pos], v16)
            for col in range(2):
                parts = []
                for bs in range(0, L, 8):
                    grp = [rowbuf[pl.ds(t * D + col * 16, 16)]
                           for t in range(bs, min(bs + 8, L))]
                    parts.append(_tree_sum(grp))
                out_buf[j, pl.ds(col * 16, 16)] = _tree_sum(parts)

    def store(out_buf, sem_o, r):
        pltpu.async_copy(
            out_buf, out_hbm.at[pl.ds(sbase + r * K, K)], sem_o)

    def wait_store(out_buf, sem_o):
        pltpu.make_async_copy(
            out_buf, out_hbm.at[pl.ds(sbase, K)], sem_o).wait()

    issue(stage_a, sem_a, 0)
    issue(stage_b, sem_b, 1)

    def body(g, _):
        ra = 2 * g
        rb = 2 * g + 1

        @pl.when(g > 0)
        def _():
            wait_store(out_a, sem_oa)
        drain(stage_a, sem_a)
        consume(stage_a, out_a)
        issue(stage_a, sem_a, ra + 2)
        store(out_a, sem_oa, ra)

        @pl.when(g > 0)
        def _():
            wait_store(out_b, sem_ob)
        drain(stage_b, sem_b)
        consume(stage_b, out_b)
        issue(stage_b, sem_b, rb + 2)
        store(out_b, sem_ob, rb)
        return 0

    lax.fori_loop(0, NR // 2, body, 0)
    wait_store(out_a, sem_oa)
    wait_store(out_b, sem_ob)


def _mlp_body(s_ref, l_ref, w1_ref, b1_ref, w2_ref, b2_ref, o_ref):
    rep = s_ref[...] * l_ref[...]
    h = lax.dot_general(rep, w1_ref[...], (((1,), (1,)), ((), ())),
                        preferred_element_type=jnp.float32) + b1_ref[...]
    h = jnp.maximum(h, 0.0)
    o_ref[...] = lax.dot_general(h, w2_ref[...], (((1,), (1,)), ((), ())),
                                 preferred_element_type=jnp.float32) + b2_ref[...]


@jax.jit
def kernel(x, lengths, table, W1, b1, W2, b2):
    xp = jnp.pad(x, ((0, 0), (0, SR - L)))
    tablef = jnp.pad(table.T, ((0, 0), (0, VPAD - VOCAB))).reshape(H * VPAD)
    sums = _sc_gather_sum(xp, tablef)
    inv_len = (1.0 / lengths.astype(jnp.float32)).reshape(B, 1)
    logits = pl.pallas_call(
        _mlp_body,
        out_shape=jax.ShapeDtypeStruct((B, C), jnp.float32),
    )(sums, inv_len, W1, b1.reshape(1, H), W2, b2.reshape(1, C))
    return logits


# speedup vs baseline: 3.8411x; 3.8411x over previous
import functools

import jax
import jax.numpy as jnp
from jax import lax
from jax.experimental import pallas as pl
from jax.experimental.pallas import tpu as pltpu
from jax.experimental.pallas import tpu_sc as plsc

VOCAB, D, H, C = 1000000, 32, 32, 10
B, L = 4096, 50

NUM_CORES = 2
NUM_SUBCORES = 16
NW = NUM_CORES * NUM_SUBCORES
LP = 56
PPW = B // (2 * NW)
LP2 = 2 * LP
K = 4
NR = PPW // K

_mesh = plsc.VectorSubcoreMesh(core_axis_name="c", subcore_axis_name="s")


def _tree_sum(vals):
    vals = list(vals)
    while len(vals) > 1:
        nxt = [vals[i] + vals[i + 1] for i in range(0, len(vals) - 1, 2)]
        if len(vals) % 2:
            nxt.append(vals[-1])
        vals = nxt
    return vals[0]


def _sum_sample(rows, j, row0, col):
    parts = []
    for bs in range(0, L, 8):
        grp = [rows[j, row0 + t, pl.ds(col * 16, 16)]
               for t in range(bs, min(bs + 8, L))]
        parts.append(_tree_sum(grp))
    return _tree_sum(parts)


@functools.partial(
    pl.kernel,
    mesh=_mesh,
    compiler_params=pltpu.CompilerParams(use_tc_tiling_on_sc=False),
    out_type=jax.ShapeDtypeStruct((B, D), jnp.float32),
    scratch_types=[
        pltpu.VMEM((PPW, LP2), jnp.int32),
        pltpu.VMEM((K, LP2, D), jnp.float32),
        pltpu.VMEM((K, LP2, D), jnp.float32),
        pltpu.VMEM((2 * K, D), jnp.float32),
        pltpu.VMEM((2 * K, D), jnp.float32),
        pltpu.SemaphoreType.DMA,
        pltpu.SemaphoreType.DMA,
        pltpu.SemaphoreType.DMA,
        pltpu.SemaphoreType.DMA,
    ],
)
def _sc_gather_sum(xp_hbm, table_hbm, out_hbm,
                   idx_v, rows_a, rows_b, out_a, out_b,
                   sem_a, sem_b, sem_oa, sem_ob):
    wid = lax.axis_index("s") * NUM_CORES + lax.axis_index("c")
    pbase = wid * PPW
    sbase = 2 * pbase
    pltpu.sync_copy(xp_hbm.at[pl.ds(pbase, PPW)], idx_v)

    def issue(buf, sem, r):
        @pl.when(r < NR)
        def _():
            for j in range(K):
                pltpu.async_copy(
                    table_hbm.at[idx_v.at[r * K + j]], buf.at[j], sem)

    def drain(buf, sem):
        for j in range(K):
            pltpu.make_async_copy(
                table_hbm.at[idx_v.at[0]], buf.at[j], sem).wait()

    def consume(buf, out_buf):
        for j in range(K):
            for h in range(2):
                out_buf[2 * j + h, pl.ds(0, 16)] = \
                    _sum_sample(buf, j, h * LP, 0)
                out_buf[2 * j + h, pl.ds(16, 16)] = \
                    _sum_sample(buf, j, h * LP, 1)

    def store(out_buf, sem_o, r):
        pltpu.async_copy(
            out_buf, out_hbm.at[pl.ds(sbase + r * 2 * K, 2 * K)], sem_o)

    def wait_store(out_buf, sem_o):
        pltpu.make_async_copy(
            out_buf, out_hbm.at[pl.ds(sbase, 2 * K)], sem_o).wait()

    issue(rows_a, sem_a, 0)
    issue(rows_b, sem_b, 1)

    def body(g, _):
        ra = 2 * g
        rb = 2 * g + 1

        @pl.when(g > 0)
        def _():
            wait_store(out_a, sem_oa)
        drain(rows_a, sem_a)
        consume(rows_a, out_a)
        issue(rows_a, sem_a, ra + 2)
        store(out_a, sem_oa, ra)

        @pl.when(g > 0)
        def _():
            wait_store(out_b, sem_ob)
        drain(rows_b, sem_b)
        consume(rows_b, out_b)
        issue(rows_b, sem_b, rb + 2)
        store(out_b, sem_ob, rb)
        return 0

    lax.fori_loop(0, NR // 2, body, 0)
    wait_store(out_a, sem_oa)
    wait_store(out_b, sem_ob)


def _mlp_body(s_ref, l_ref, w1_ref, b1_ref, w2_ref, b2_ref, o_ref):
    rep = s_ref[...] * l_ref[...]
    h = lax.dot_general(rep, w1_ref[...], (((1,), (1,)), ((), ())),
                        preferred_element_type=jnp.float32) + b1_ref[...]
    h = jnp.maximum(h, 0.0)
    o_ref[...] = lax.dot_general(h, w2_ref[...], (((1,), (1,)), ((), ())),
                                 preferred_element_type=jnp.float32) + b2_ref[...]


@jax.jit
def kernel(x, lengths, table, W1, b1, W2, b2):
    xp = jnp.pad(x, ((0, 0), (0, LP - L))).reshape(B // 2, LP2)
    sums = _sc_gather_sum(xp, table)
    inv_len = (1.0 / lengths.astype(jnp.float32)).reshape(B, 1)
    logits = pl.pallas_call(
        _mlp_body,
        out_shape=jax.ShapeDtypeStruct((B, C), jnp.float32),
    )(sums, inv_len, W1, b1.reshape(1, H), W2, b2.reshape(1, C))
    return logits
